# baseline (speedup 1.0000x reference)
import functools

import jax
import jax.numpy as jnp
from jax import lax
from jax.experimental import pallas as pl
from jax.experimental.pallas import tpu as pltpu
from jax.experimental.pallas import tpu_sc as plsc

VOCAB = 1_000_000
EMBED_DIM = 32
BATCH = 16384
HIST = 200


_C = 83328
_N_CHUNK = 12
_MAIN = _C * _N_CHUNK


_NBUF = 3


def _matvec_body(wt_ref, b_ref, embt_hbm, s_ref, buf, sem):
    i = pl.program_id(0)

    def start(step):
        slot = lax.rem(step, _NBUF)
        for r in range(EMBED_DIM // 8):
            pltpu.make_async_copy(
                embt_hbm.at[pl.ds(8 * r, 8), pl.ds(step * _C, _C)],
                buf.at[slot, pl.ds(8 * r, 8)], sem.at[slot, r]
            ).start()

    @pl.when(i == 0)
    def _():
        for k in range(_NBUF - 1):
            start(k)

    @pl.when(i + _NBUF - 1 < _N_CHUNK)
    def _():
        start(i + _NBUF - 1)

    slot = lax.rem(i, _NBUF)
    for r in range(EMBED_DIM // 8):
        pltpu.make_async_copy(
            embt_hbm.at[pl.ds(8 * r, 8), pl.ds(i * _C, _C)],
            buf.at[slot, pl.ds(8 * r, 8)], sem.at[slot, r]
        ).wait()
    vals = buf[slot] * wt_ref[...]
    red = jnp.sum(vals, axis=0) + b_ref[0, 0]
    r = lax.rem(i, _SUB)
    s_ref[pl.ds(r, 1), :] = red[None, :]


_SUB = 8


def _table_scores(emb_t, wt_scaled, b_scaled):
    s2d = pl.pallas_call(
        _matvec_body,
        grid=(_N_CHUNK,),
        in_specs=[
            pl.BlockSpec((EMBED_DIM, 1), lambda i: (0, 0)),
            pl.BlockSpec(memory_space=pltpu.SMEM),
            pl.BlockSpec(memory_space=pl.ANY),
        ],
        out_specs=pl.BlockSpec((_SUB, _C), lambda i: (i // _SUB, 0)),
        out_shape=jax.ShapeDtypeStruct((2 * _SUB, _C), jnp.float32),
        scratch_shapes=[
            pltpu.VMEM((_NBUF, EMBED_DIM, _C), jnp.float32),
            pltpu.SemaphoreType.DMA((_NBUF, EMBED_DIM // 8)),
        ],
        compiler_params=pltpu.CompilerParams(
            dimension_semantics=("arbitrary",)),
    )(wt_scaled, b_scaled, emb_t)
    return s2d.reshape(2 * _SUB * _C)[:_MAIN]


_NW = 32
_ROWS_PER_W = BATCH // _NW
_GRP = 16
_N_GRP = _ROWS_PER_W // _GRP
_IDX_PER_GRP = _GRP * HIST


_UNROLL = 8


_NPIPE = 4


def _bag_kernel(tok_hbm, s_hbm, out_hbm, tok_v0, tok_v1, tok_v2, tok_v3,
                val_v0, val_v1, val_v2, val_v3, out_v, tsem, vsem):
    nc = 2
    wid = lax.axis_index("s") * nc + lax.axis_index("c")
    base = wid * (_ROWS_PER_W * HIST)
    lane = lax.iota(jnp.int32, 16) * HIST
    tok_b = (tok_v0, tok_v1, tok_v2, tok_v3)
    val_b = (val_v0, val_v1, val_v2, val_v3)

    def tok_start(g):
        return pltpu.async_copy(
            tok_hbm.at[pl.ds(base + g * _IDX_PER_GRP, _IDX_PER_GRP)],
            tok_b[g % _NPIPE], tsem.at[g % _NPIPE])

    def tok_wait(g):
        pltpu.make_async_copy(
            tok_hbm.at[pl.ds(base + g * _IDX_PER_GRP, _IDX_PER_GRP)],
            tok_b[g % _NPIPE], tsem.at[g % _NPIPE]).wait()

    def gat_start(g):
        return pltpu.async_copy(s_hbm.at[tok_b[g % _NPIPE]],
                                val_b[g % _NPIPE], vsem.at[g % _NPIPE])

    def gat_wait(g):
        pltpu.make_async_copy(s_hbm.at[tok_b[g % _NPIPE]],
                              val_b[g % _NPIPE], vsem.at[g % _NPIPE]).wait()

    tok_start(0)
    tok_start(1)
    tok_start(2)
    tok_start(3)
    tok_wait(0)
    gat_start(0)
    tok_wait(1)
    gat_start(1)
    tok_wait(2)
    gat_start(2)

    for g in range(_N_GRP):
        gat_wait(g)
        if g + 3 < _N_GRP:
            tok_wait(g + 3)
            gat_start(g + 3)
        if g + 4 < _N_GRP:
            tok_start(g + 4)

        vv = val_b[g % _NPIPE]

        def body(i, acc):
            n = i * _UNROLL
            for u in range(_UNROLL):
                acc = acc + plsc.load_gather(vv, [lane + (n + u)])
            return acc

        acc = lax.fori_loop(0, HIST // _UNROLL, body,
                            jnp.zeros((16,), jnp.float32))
        out_v[pl.ds(g * _GRP, _GRP)] = acc

    pltpu.sync_copy(out_v, out_hbm.at[pl.ds(wid * _ROWS_PER_W, _ROWS_PER_W)])


@functools.cache
def _bag():
    return pl.kernel(
        _bag_kernel,
        mesh=plsc.VectorSubcoreMesh(core_axis_name="c", subcore_axis_name="s"),
        compiler_params=pltpu.CompilerParams(needs_layout_passes=False),
        out_type=jax.ShapeDtypeStruct((BATCH,), jnp.float32),
        scratch_types=[
            pltpu.VMEM((_IDX_PER_GRP,), jnp.int32),
            pltpu.VMEM((_IDX_PER_GRP,), jnp.int32),
            pltpu.VMEM((_IDX_PER_GRP,), jnp.int32),
            pltpu.VMEM((_IDX_PER_GRP,), jnp.int32),
            pltpu.VMEM((_IDX_PER_GRP,), jnp.float32),
            pltpu.VMEM((_IDX_PER_GRP,), jnp.float32),
            pltpu.VMEM((_IDX_PER_GRP,), jnp.float32),
            pltpu.VMEM((_IDX_PER_GRP,), jnp.float32),
            pltpu.VMEM((_ROWS_PER_W,), jnp.float32),
            pltpu.SemaphoreType.DMA((_NPIPE,)),
            pltpu.SemaphoreType.DMA((_NPIPE,)),
        ],
    )


def kernel(text_tokens, emb_weight, fc_weight, fc_bias):
    inv = jnp.float32(1.0 / HIST)
    wt_scaled = fc_weight.astype(jnp.float32).T * inv
    b_scaled = (fc_bias.astype(jnp.float32) * inv).reshape(1, 1)
    s_main = _table_scores(emb_weight.T, wt_scaled, b_scaled)
    s_tail = emb_weight[_MAIN:] @ wt_scaled[:, 0] + fc_bias[0] * inv
    s = jnp.concatenate([s_main, s_tail])
    tok_flat = text_tokens.astype(jnp.int32).reshape(BATCH * HIST)
    out = _bag()(tok_flat, s)
    return out.reshape(BATCH, 1)


# speedup vs baseline: 1.0098x; 1.0098x over previous
import functools

import jax
import jax.numpy as jnp
from jax import lax
from jax.experimental import pallas as pl
from jax.experimental.pallas import tpu as pltpu
from jax.experimental.pallas import tpu_sc as plsc

VOCAB = 1_000_000
EMBED_DIM = 32
BATCH = 16384
HIST = 200


_C = 32256
_N_CHUNK = 31
_MAIN = _C * _N_CHUNK


_NBUF = 4


def _matvec_body(wt_ref, b_ref, embt_hbm, s_ref, buf, sem):
    i = pl.program_id(0)

    def start(step):
        slot = lax.rem(step, _NBUF)
        for r in range(EMBED_DIM // 8):
            pltpu.make_async_copy(
                embt_hbm.at[pl.ds(8 * r, 8), pl.ds(step * _C, _C)],
                buf.at[slot, pl.ds(8 * r, 8)], sem.at[slot, r]
            ).start()

    @pl.when(i == 0)
    def _():
        for k in range(_NBUF - 1):
            start(k)

    @pl.when(i + _NBUF - 1 < _N_CHUNK)
    def _():
        start(i + _NBUF - 1)

    slot = lax.rem(i, _NBUF)
    for r in range(EMBED_DIM // 8):
        pltpu.make_async_copy(
            embt_hbm.at[pl.ds(8 * r, 8), pl.ds(i * _C, _C)],
            buf.at[slot, pl.ds(8 * r, 8)], sem.at[slot, r]
        ).wait()
    vals = buf[slot] * wt_ref[...]
    red = jnp.sum(vals, axis=0) + b_ref[0, 0]
    r = lax.rem(i, _SUB)
    s_ref[pl.ds(r, 1), :] = red[None, :]


_SUB = 8


def _table_scores(emb_t, wt_scaled, b_scaled):
    s2d = pl.pallas_call(
        _matvec_body,
        grid=(_N_CHUNK,),
        in_specs=[
            pl.BlockSpec((EMBED_DIM, 1), lambda i: (0, 0)),
            pl.BlockSpec(memory_space=pltpu.SMEM),
            pl.BlockSpec(memory_space=pl.ANY),
        ],
        out_specs=pl.BlockSpec((_SUB, _C), lambda i: (i // _SUB, 0)),
        out_shape=jax.ShapeDtypeStruct((4 * _SUB, _C), jnp.float32),
        scratch_shapes=[
            pltpu.VMEM((_NBUF, EMBED_DIM, _C), jnp.float32),
            pltpu.SemaphoreType.DMA((_NBUF, EMBED_DIM // 8)),
        ],
        compiler_params=pltpu.CompilerParams(
            dimension_semantics=("arbitrary",)),
    )(wt_scaled, b_scaled, emb_t)
    return s2d.reshape(4 * _SUB * _C)[:_MAIN]


_NW = 32
_ROWS_PER_W = BATCH // _NW
_GRP = 16
_N_GRP = _ROWS_PER_W // _GRP
_IDX_PER_GRP = _GRP * HIST


_UNROLL = 8


_NPIPE = 4


def _bag_kernel(tok_hbm, s_hbm, out_hbm, tok_v0, tok_v1, tok_v2, tok_v3,
                val_v0, val_v1, val_v2, val_v3, out_v, tsem, vsem):
    nc = 2
    wid = lax.axis_index("s") * nc + lax.axis_index("c")
    base = wid * (_ROWS_PER_W * HIST)
    lane = lax.iota(jnp.int32, 16) * HIST
    tok_b = (tok_v0, tok_v1, tok_v2, tok_v3)
    val_b = (val_v0, val_v1, val_v2, val_v3)

    def tok_start(g):
        return pltpu.async_copy(
            tok_hbm.at[pl.ds(base + g * _IDX_PER_GRP, _IDX_PER_GRP)],
            tok_b[g % _NPIPE], tsem.at[g % _NPIPE])

    def tok_wait(g):
        pltpu.make_async_copy(
            tok_hbm.at[pl.ds(base + g * _IDX_PER_GRP, _IDX_PER_GRP)],
            tok_b[g % _NPIPE], tsem.at[g % _NPIPE]).wait()

    def gat_start(g):
        return pltpu.async_copy(s_hbm.at[tok_b[g % _NPIPE]],
                                val_b[g % _NPIPE], vsem.at[g % _NPIPE])

    def gat_wait(g):
        pltpu.make_async_copy(s_hbm.at[tok_b[g % _NPIPE]],
                              val_b[g % _NPIPE], vsem.at[g % _NPIPE]).wait()

    tok_start(0)
    tok_start(1)
    tok_start(2)
    tok_start(3)
    tok_wait(0)
    gat_start(0)
    tok_wait(1)
    gat_start(1)
    tok_wait(2)
    gat_start(2)

    for g in range(_N_GRP):
        gat_wait(g)
        if g + 3 < _N_GRP:
            tok_wait(g + 3)
            gat_start(g + 3)
        if g + 4 < _N_GRP:
            tok_start(g + 4)

        vv = val_b[g % _NPIPE]

        def body(i, acc):
            n = i * _UNROLL
            for u in range(_UNROLL):
                acc = acc + plsc.load_gather(vv, [lane + (n + u)])
            return acc

        acc = lax.fori_loop(0, HIST // _UNROLL, body,
                            jnp.zeros((16,), jnp.float32))
        out_v[pl.ds(g * _GRP, _GRP)] = acc

    pltpu.sync_copy(out_v, out_hbm.at[pl.ds(wid * _ROWS_PER_W, _ROWS_PER_W)])


@functools.cache
def _bag():
    return pl.kernel(
        _bag_kernel,
        mesh=plsc.VectorSubcoreMesh(core_axis_name="c", subcore_axis_name="s"),
        compiler_params=pltpu.CompilerParams(needs_layout_passes=False),
        out_type=jax.ShapeDtypeStruct((BATCH,), jnp.float32),
        scratch_types=[
            pltpu.VMEM((_IDX_PER_GRP,), jnp.int32),
            pltpu.VMEM((_IDX_PER_GRP,), jnp.int32),
            pltpu.VMEM((_IDX_PER_GRP,), jnp.int32),
            pltpu.VMEM((_IDX_PER_GRP,), jnp.int32),
            pltpu.VMEM((_IDX_PER_GRP,), jnp.float32),
            pltpu.VMEM((_IDX_PER_GRP,), jnp.float32),
            pltpu.VMEM((_IDX_PER_GRP,), jnp.float32),
            pltpu.VMEM((_IDX_PER_GRP,), jnp.float32),
            pltpu.VMEM((_ROWS_PER_W,), jnp.float32),
            pltpu.SemaphoreType.DMA((_NPIPE,)),
            pltpu.SemaphoreType.DMA((_NPIPE,)),
        ],
    )


def kernel(text_tokens, emb_weight, fc_weight, fc_bias):
    inv = jnp.float32(1.0 / HIST)
    wt_scaled = fc_weight.astype(jnp.float32).T * inv
    b_scaled = (fc_bias.astype(jnp.float32) * inv).reshape(1, 1)
    s_main = _table_scores(emb_weight.T, wt_scaled, b_scaled)
    s_tail = emb_weight[_MAIN:] @ wt_scaled[:, 0] + fc_bias[0] * inv
    s = jnp.concatenate([s_main, s_tail])
    tok_flat = text_tokens.astype(jnp.int32).reshape(BATCH * HIST)
    out = _bag()(tok_flat, s)
    return out.reshape(BATCH, 1)


# speedup vs baseline: 1.0163x; 1.0064x over previous
import functools

import jax
import jax.numpy as jnp
from jax import lax
from jax.experimental import pallas as pl
from jax.experimental.pallas import tpu as pltpu
from jax.experimental.pallas import tpu_sc as plsc

VOCAB = 1_000_000
EMBED_DIM = 32
BATCH = 16384
HIST = 200


_C = 32256
_N_CHUNK = 31
_MAIN = _C * _N_CHUNK


_NBUF = 4


def _matvec_body(wt_ref, b_ref, embt_hbm, s_ref, buf, sem):
    i = pl.program_id(0)

    def start(step):
        slot = lax.rem(step, _NBUF)
        for r in range(EMBED_DIM // 8):
            pltpu.make_async_copy(
                embt_hbm.at[pl.ds(8 * r, 8), pl.ds(step * _C, _C)],
                buf.at[slot, pl.ds(8 * r, 8)], sem.at[slot, r]
            ).start()

    @pl.when(i == 0)
    def _():
        for k in range(_NBUF - 1):
            start(k)

    @pl.when(i + _NBUF - 1 < _N_CHUNK)
    def _():
        start(i + _NBUF - 1)

    slot = lax.rem(i, _NBUF)
    for r in range(EMBED_DIM // 8):
        pltpu.make_async_copy(
            embt_hbm.at[pl.ds(8 * r, 8), pl.ds(i * _C, _C)],
            buf.at[slot, pl.ds(8 * r, 8)], sem.at[slot, r]
        ).wait()
    vals = buf[slot] * wt_ref[...]
    red = jnp.sum(vals, axis=0) + b_ref[0, 0]
    r = lax.rem(i, _SUB)
    s_ref[pl.ds(r, 1), :] = red[None, :]


_SUB = 8


def _table_scores(emb_t, wt_scaled, b_scaled):
    s2d = pl.pallas_call(
        _matvec_body,
        grid=(_N_CHUNK,),
        in_specs=[
            pl.BlockSpec((EMBED_DIM, 1), lambda i: (0, 0)),
            pl.BlockSpec(memory_space=pltpu.SMEM),
            pl.BlockSpec(memory_space=pl.ANY),
        ],
        out_specs=pl.BlockSpec((_SUB, _C), lambda i: (i // _SUB, 0)),
        out_shape=jax.ShapeDtypeStruct((4 * _SUB, _C), jnp.float32),
        scratch_shapes=[
            pltpu.VMEM((_NBUF, EMBED_DIM, _C), jnp.float32),
            pltpu.SemaphoreType.DMA((_NBUF, EMBED_DIM // 8)),
        ],
        compiler_params=pltpu.CompilerParams(
            dimension_semantics=("arbitrary",)),
    )(wt_scaled, b_scaled, emb_t)
    return s2d.reshape(4 * _SUB * _C)[:_MAIN]


_NW = 32
_ROWS_PER_W = BATCH // _NW
_GRP = 32
_N_GRP = _ROWS_PER_W // _GRP
_IDX_PER_GRP = _GRP * HIST


_UNROLL = 8


_NPIPE = 4


def _bag_kernel(tok_hbm, s_hbm, out_hbm, tok_v0, tok_v1, tok_v2, tok_v3,
                val_v0, val_v1, val_v2, val_v3, out_v, tsem, vsem):
    nc = 2
    wid = lax.axis_index("s") * nc + lax.axis_index("c")
    base = wid * (_ROWS_PER_W * HIST)
    lane = lax.iota(jnp.int32, 16) * HIST
    tok_b = (tok_v0, tok_v1, tok_v2, tok_v3)
    val_b = (val_v0, val_v1, val_v2, val_v3)

    def tok_start(g):
        return pltpu.async_copy(
            tok_hbm.at[pl.ds(base + g * _IDX_PER_GRP, _IDX_PER_GRP)],
            tok_b[g % _NPIPE], tsem.at[g % _NPIPE])

    def tok_wait(g):
        pltpu.make_async_copy(
            tok_hbm.at[pl.ds(base + g * _IDX_PER_GRP, _IDX_PER_GRP)],
            tok_b[g % _NPIPE], tsem.at[g % _NPIPE]).wait()

    def gat_start(g):
        return pltpu.async_copy(s_hbm.at[tok_b[g % _NPIPE]],
                                val_b[g % _NPIPE], vsem.at[g % _NPIPE])

    def gat_wait(g):
        pltpu.make_async_copy(s_hbm.at[tok_b[g % _NPIPE]],
                              val_b[g % _NPIPE], vsem.at[g % _NPIPE]).wait()

    tok_start(0)
    tok_start(1)
    tok_start(2)
    tok_start(3)
    tok_wait(0)
    gat_start(0)
    tok_wait(1)
    gat_start(1)
    tok_wait(2)
    gat_start(2)

    for g in range(_N_GRP):
        gat_wait(g)
        if g + 3 < _N_GRP:
            tok_wait(g + 3)
            gat_start(g + 3)
        if g + 4 < _N_GRP:
            tok_start(g + 4)

        vv = val_b[g % _NPIPE]

        def body(i, accs):
            a0, a1 = accs
            n = i * _UNROLL
            for u in range(_UNROLL):
                a0 = a0 + plsc.load_gather(vv, [lane + (n + u)])
                a1 = a1 + plsc.load_gather(vv, [lane + (16 * HIST + n + u)])
            return a0, a1

        z = jnp.zeros((16,), jnp.float32)
        a0, a1 = lax.fori_loop(0, HIST // _UNROLL, body, (z, z))
        out_v[pl.ds(g * _GRP, 16)] = a0
        out_v[pl.ds(g * _GRP + 16, 16)] = a1

    pltpu.sync_copy(out_v, out_hbm.at[pl.ds(wid * _ROWS_PER_W, _ROWS_PER_W)])


@functools.cache
def _bag():
    return pl.kernel(
        _bag_kernel,
        mesh=plsc.VectorSubcoreMesh(core_axis_name="c", subcore_axis_name="s"),
        compiler_params=pltpu.CompilerParams(needs_layout_passes=False),
        out_type=jax.ShapeDtypeStruct((BATCH,), jnp.float32),
        scratch_types=[
            pltpu.VMEM((_IDX_PER_GRP,), jnp.int32),
            pltpu.VMEM((_IDX_PER_GRP,), jnp.int32),
            pltpu.VMEM((_IDX_PER_GRP,), jnp.int32),
            pltpu.VMEM((_IDX_PER_GRP,), jnp.int32),
            pltpu.VMEM((_IDX_PER_GRP,), jnp.float32),
            pltpu.VMEM((_IDX_PER_GRP,), jnp.float32),
            pltpu.VMEM((_IDX_PER_GRP,), jnp.float32),
            pltpu.VMEM((_IDX_PER_GRP,), jnp.float32),
            pltpu.VMEM((_ROWS_PER_W,), jnp.float32),
            pltpu.SemaphoreType.DMA((_NPIPE,)),
            pltpu.SemaphoreType.DMA((_NPIPE,)),
        ],
    )


def kernel(text_tokens, emb_weight, fc_weight, fc_bias):
    inv = jnp.float32(1.0 / HIST)
    wt_scaled = fc_weight.astype(jnp.float32).T * inv
    b_scaled = (fc_bias.astype(jnp.float32) * inv).reshape(1, 1)
    s_main = _table_scores(emb_weight.T, wt_scaled, b_scaled)
    s_tail = emb_weight[_MAIN:] @ wt_scaled[:, 0] + fc_bias[0] * inv
    s = jnp.concatenate([s_main, s_tail])
    tok_flat = text_tokens.astype(jnp.int32).reshape(BATCH * HIST)
    out = _bag()(tok_flat, s)
    return out.reshape(BATCH, 1)
